# Initial kernel scaffold; baseline (speedup 1.0000x reference)
import jax
import jax.numpy as jnp
from jax.experimental import pallas as pl


def kernel(x, weight, bias):
    raise NotImplementedError("write your pallas kernel here")


# speedup vs baseline: 1.2752x; 1.2752x over previous
import functools

import jax
import jax.numpy as jnp
from jax.experimental import pallas as pl
from jax.experimental.pallas import tpu as pltpu

N_EXPERTS = 64
K = 8
BLK = 512


def _gate_body(x_ref, w_ref, b_ref, wout_ref, iout_ref):
    scores = jax.lax.dot_general(
        x_ref[...], w_ref[...],
        dimension_numbers=(((1,), (1,)), ((), ())),
        preferred_element_type=jnp.float32,
        precision=jax.lax.Precision.DEFAULT,
    )
    scores = scores + b_ref[...]

    col = jax.lax.broadcasted_iota(jnp.int32, scores.shape, 1)
    s = scores
    vals, ids = [], []
    for _ in range(K):
        a = jnp.argmax(s, axis=1)
        m = jnp.max(s, axis=1)
        vals.append(m)
        ids.append(a)
        s = jnp.where(col == a[:, None], -jnp.inf, s)
    tv = jnp.stack(vals, axis=1)
    ti = jnp.stack(ids, axis=1)

    e = jnp.exp(tv - tv[:, 0:1])
    w = e / jnp.sum(e, axis=1, keepdims=True)
    wout_ref[...] = w
    iout_ref[...] = ti


@jax.jit
def kernel(x, weight, bias):
    n_rows = x.shape[0]
    grid = (n_rows // BLK,)
    wout, iout = pl.pallas_call(
        _gate_body,
        grid=grid,
        in_specs=[
            pl.BlockSpec((BLK, x.shape[1]), lambda i: (i, 0)),
            pl.BlockSpec((N_EXPERTS, x.shape[1]), lambda i: (0, 0)),
            pl.BlockSpec((1, N_EXPERTS), lambda i: (0, 0)),
        ],
        out_specs=[
            pl.BlockSpec((BLK, K), lambda i: (i, 0)),
            pl.BlockSpec((BLK, K), lambda i: (i, 0)),
        ],
        out_shape=[
            jax.ShapeDtypeStruct((n_rows, K), jnp.float32),
            jax.ShapeDtypeStruct((n_rows, K), jnp.int32),
        ],
    )(x, weight, bias.reshape(1, N_EXPERTS))
    return wout, iout
